# Initial kernel scaffold; baseline (speedup 1.0000x reference)
import jax
import jax.numpy as jnp
from jax.experimental import pallas as pl


def kernel(idx1, image_in1, image_out1, idx2, image_in2, image_out2, idx3, image_in3, image_out3, idx4, image_in4, image_out4):
    raise NotImplementedError("write your pallas kernel here")


# speedup vs baseline: 2.3803x; 2.3803x over previous
import functools

import jax
import jax.numpy as jnp
from jax import lax
from jax.experimental import pallas as pl
from jax.experimental.pallas import tpu as pltpu
from jax.experimental.pallas import tpu_sc as plsc

BATCH = 1024
IMG = 64 * 64
ROWS = 128
LANES = 16


def _tc_body(in1, out1, in2, out2, in3, out3, in4, out4, o_ref):
    for g, (a, b) in enumerate(((in1, out1), (in2, out2), (in3, out3), (in4, out4))):
        d = b[...] - a[...]
        o_ref[g, :] = jnp.sum(d * d, axis=1)


def _tc_per_sample(i1, o1, i2, o2, i3, o3, i4, o4):
    grid = BATCH // ROWS
    img_spec = pl.BlockSpec((ROWS, IMG), lambda i: (i, 0))
    return pl.pallas_call(
        _tc_body,
        grid=(grid,),
        in_specs=[img_spec] * 8,
        out_specs=pl.BlockSpec((4, ROWS), lambda i: (0, i)),
        out_shape=jax.ShapeDtypeStruct((4, BATCH), jnp.float32),
    )(i1, o1, i2, o2, i3, o3, i4, o4)


def _sc_body(idx_hbm, s_hbm, o_hbm, idx_v, s_v, acc_v, cnt_v, res_v):
    nvec = BATCH // LANES

    @pl.when((lax.axis_index("c") == 0) & (lax.axis_index("s") == 0))
    def _():
        zero = jnp.zeros((LANES,), jnp.float32)

        def zloop(i, _):
            acc_v[pl.ds(i * LANES, LANES)] = zero
            cnt_v[pl.ds(i * LANES, LANES)] = zero
            return 0

        lax.fori_loop(0, nvec, zloop, 0)

        ones = jnp.ones((LANES,), jnp.float32)
        for g in range(4):
            pltpu.sync_copy(idx_hbm.at[g], idx_v)
            pltpu.sync_copy(s_hbm.at[g], s_v)

            def sloop(i, _):
                iv = idx_v[pl.ds(i * LANES, LANES)]
                sv = s_v[pl.ds(i * LANES, LANES)]
                plsc.addupdate_scatter(acc_v, [iv], sv)
                plsc.addupdate_scatter(cnt_v, [iv], ones)
                return 0

            lax.fori_loop(0, nvec, sloop, 0)

        def rloop(i, t):
            a = acc_v[pl.ds(i * LANES, LANES)]
            c = cnt_v[pl.ds(i * LANES, LANES)]
            return t + a / c

        tot = lax.fori_loop(0, nvec, rloop, jnp.zeros((LANES,), jnp.float32))
        mean = lax.reduce_sum_p.bind(tot, axes=(0,)) * jnp.float32(1.0 / BATCH)
        res_v[...] = jnp.full((LANES,), mean, jnp.float32)
        pltpu.sync_copy(res_v, o_hbm)


def _sc_accum(idx4, s4):
    mesh = plsc.VectorSubcoreMesh(core_axis_name="c", subcore_axis_name="s")
    f = pl.kernel(
        _sc_body,
        out_type=jax.ShapeDtypeStruct((LANES,), jnp.float32),
        mesh=mesh,
        compiler_params=pltpu.CompilerParams(needs_layout_passes=False),
        scratch_types=[
            pltpu.VMEM((BATCH,), jnp.int32),
            pltpu.VMEM((BATCH,), jnp.float32),
            pltpu.VMEM((BATCH,), jnp.float32),
            pltpu.VMEM((BATCH,), jnp.float32),
            pltpu.VMEM((LANES,), jnp.float32),
        ],
    )
    return f(idx4, s4)


def kernel(idx1, image_in1, image_out1, idx2, image_in2, image_out2,
           idx3, image_in3, image_out3, idx4, image_in4, image_out4):
    imgs = [a.reshape(BATCH, IMG) for a in
            (image_in1, image_out1, image_in2, image_out2,
             image_in3, image_out3, image_in4, image_out4)]
    s = _tc_per_sample(*imgs)
    idx4 = jnp.stack([idx1.astype(jnp.int32), idx2.astype(jnp.int32),
                      idx3.astype(jnp.int32), idx4.astype(jnp.int32)])
    out = _sc_accum(idx4, s)
    return out[0]
